# Initial kernel scaffold; baseline (speedup 1.0000x reference)
import jax
import jax.numpy as jnp
from jax.experimental import pallas as pl


def kernel(input_ids, puzzle_identifiers, tok_table, puzzle_table):
    raise NotImplementedError("write your pallas kernel here")


# speedup vs baseline: 1.7576x; 1.7576x over previous
import functools

import jax
import jax.numpy as jnp
from jax import lax
from jax.experimental import pallas as pl
from jax.experimental.pallas import tpu as pltpu
from jax.experimental.pallas import tpu_sc as plsc

_B = 4
_S = 8192
_D = 1024
_N = _B * _S
_NC = 2
_NS = 16
_NW = _NC * _NS
_RPW = _N // _NW
_CH = 32
_NBUF = 2
_NCH = _RPW // _CH
_POS = 1

_mesh = plsc.VectorSubcoreMesh(
    core_axis_name="c", subcore_axis_name="s", num_cores=_NC, num_subcores=_NS
)


@functools.partial(
    pl.kernel,
    out_type=jax.ShapeDtypeStruct((_N, _D), jnp.float32),
    mesh=_mesh,
    scratch_types=[
        pltpu.VMEM((_RPW,), jnp.int32),
        pltpu.VMEM((_B, 1), jnp.int32),
        pltpu.VMEM((_NBUF, _CH, _D), jnp.float32),
        pltpu.VMEM((1, _D), jnp.float32),
        pltpu.SemaphoreType.DMA,
        pltpu.SemaphoreType.DMA,
        pltpu.SemaphoreType.DMA,
        pltpu.SemaphoreType.DMA,
        pltpu.SemaphoreType.DMA,
    ],
)
def _emb_lookup(tok_hbm, puz_hbm, ids_hbm, pid_hbm, out_hbm,
                idx_v, pid_v, bufs, prow_v,
                sem_in0, sem_in1, sem_out0, sem_out1, sem_p):
    sems_in = (sem_in0, sem_in1)
    sems_out = (sem_out0, sem_out1)
    wid = lax.axis_index("s") * _NC + lax.axis_index("c")
    base = pl.multiple_of(wid * _RPW, _RPW)

    pltpu.sync_copy(ids_hbm.at[pl.ds(base, _RPW)], idx_v)
    pltpu.sync_copy(pid_hbm, pid_v)

    def _gather(ch, b):
        off = pl.multiple_of(ch * _CH, _CH)
        return pltpu.make_async_copy(
            tok_hbm.at[idx_v.at[pl.ds(off, _CH)]], bufs.at[b], sems_in[b]
        )

    def _writeout(ch, b):
        off = pl.multiple_of(ch * _CH, _CH)
        return pltpu.make_async_copy(
            bufs.at[b], out_hbm.at[pl.ds(base + off, _CH)], sems_out[b]
        )

    for b in range(_NBUF):
        _gather(b, b).start()

    def _step(o, _):
        g0 = o * _NBUF
        for b in range(_NBUF):
            g = g0 + b
            _gather(g, b).wait()
            _writeout(g, b).start()
            _writeout(g, b).wait()

            @pl.when(o + 1 < _NCH // _NBUF)
            def _():
                _gather(g + _NBUF, b).start()

        return 0

    lax.fori_loop(0, _NCH // _NBUF, _step, 0)

    def _patch(bb):
        cp = pltpu.make_async_copy(
            puz_hbm.at[pid_v.at[bb]], prow_v, sem_p
        )
        cp.start()
        cp.wait()
        pltpu.sync_copy(prow_v, out_hbm.at[pl.ds(base + _POS, 1)])

    for w in range(_B):

        @pl.when(wid == w * (_NW // _B))
        def _(bb=w):
            _patch(bb)


def kernel(input_ids, puzzle_identifiers, tok_table, puzzle_table):
    ids = input_ids.reshape(_N).astype(jnp.int32)
    pid = puzzle_identifiers.reshape(_B, 1).astype(jnp.int32)
    out = _emb_lookup(tok_table, puzzle_table, ids, pid)
    return out.reshape(_B, _S, _D)
